# baseline (device time: 192459 ns/iter reference)
import jax
import jax.numpy as jnp
from jax import lax
from jax.experimental import pallas as pl
from jax.experimental.pallas import tpu as pltpu

QROWS = 2048
CH = 256
NCK = QROWS // CH

DIAG_X = ((0, 336), (336, 336))
FWD_Y = (((672, 344), (2, 3)), ((1016, 344), (3, 4, 5)))
FWD_Z = (((1360, 344), (5, 6)), ((1704, 344), (6, 7)))
NDIAG_X = len(DIAG_X)


def kernel(x):
    m, n = x.shape

    def body(
        x_ref, out_ref, bown_ref, bdiag_ref,
        sx_send, sx_recv,
        s2_send, s2_recv,
        s3_send, s3_recv,
        s4_send, s4_recv,
        s5_send, s5_recv,
        cp_sems,
    ):
        my_x = lax.axis_index("x")
        my_y = lax.axis_index("y")
        my_z = lax.axis_index("z")
        xn = (1 - my_x, my_y, my_z)
        yn = (my_x, 1 - my_y, my_z)
        zn = (my_x, my_y, 1 - my_z)

        q = 2 * my_y + my_z
        yq = 2 * (1 - my_y) + my_z
        zq = 2 * my_y + (1 - my_z)
        dq = 3 - q

        def rows(quarter, k):
            return pl.ds(quarter * QROWS + k * CH, CH)

        def rowsr(quarter, r0, nr):
            return pl.ds(quarter * QROWS + r0, nr)

        barrier_sem = pltpu.get_barrier_semaphore()
        for nbr in (xn, yn, zn):
            pl.semaphore_signal(
                barrier_sem, inc=1,
                device_id=nbr, device_id_type=pl.DeviceIdType.MESH,
            )
        pl.semaphore_wait(barrier_sem, 3)

        f1 = []
        for k in range(NCK):
            r = pltpu.make_async_remote_copy(
                src_ref=x_ref.at[rows(q, k), :],
                dst_ref=bown_ref.at[pl.ds(k * CH, CH), :],
                send_sem=sx_send.at[k],
                recv_sem=sx_recv.at[k],
                device_id=xn,
                device_id_type=pl.DeviceIdType.MESH,
            )
            r.start()
            f1.append(r)
        for j, (r0, nr) in enumerate(DIAG_X):
            r = pltpu.make_async_remote_copy(
                src_ref=x_ref.at[rowsr(dq, r0, nr), :],
                dst_ref=bdiag_ref.at[pl.ds(r0, nr), :],
                send_sem=sx_send.at[NCK + j],
                recv_sem=sx_recv.at[NCK + j],
                device_id=xn,
                device_id_type=pl.DeviceIdType.MESH,
            )
            r.start()
            f1.append(r)

        f2, f3, cps = [], [], []
        for k in range(NCK):
            f1[k].wait_recv()
            ksl = pl.ds(k * CH, CH)
            bown_ref[ksl, :] = bown_ref[ksl, :] + x_ref[rows(q, k), :]
            for sems, lst, nbr in (
                ((s2_send, s2_recv), f2, yn),
                ((s3_send, s3_recv), f3, zn),
            ):
                r = pltpu.make_async_remote_copy(
                    src_ref=bown_ref.at[ksl, :],
                    dst_ref=out_ref.at[rows(q, k), :],
                    send_sem=sems[0].at[k],
                    recv_sem=sems[1].at[k],
                    device_id=nbr,
                    device_id_type=pl.DeviceIdType.MESH,
                )
                r.start()
                lst.append(r)
            cp = pltpu.make_async_copy(
                bown_ref.at[ksl, :], out_ref.at[rows(q, k), :], cp_sems.at[k]
            )
            cp.start()
            cps.append(cp)

        for j, (r0, nr) in enumerate(DIAG_X):
            f1[NCK + j].wait_recv()
            jsl = pl.ds(r0, nr)
            bdiag_ref[jsl, :] = bdiag_ref[jsl, :] + x_ref[rowsr(dq, r0, nr), :]
            cp = pltpu.make_async_copy(
                bdiag_ref.at[jsl, :], out_ref.at[rowsr(dq, r0, nr), :],
                cp_sems.at[NCK + j],
            )
            cp.start()
            cps.append(cp)

        def recv_mirror(quarter, k, recv_sem, send_sem):
            return pltpu.make_async_remote_copy(
                src_ref=bown_ref.at[pl.ds(k * CH, CH), :],
                dst_ref=out_ref.at[rows(quarter, k), :],
                send_sem=send_sem,
                recv_sem=recv_sem,
                device_id=xn,
                device_id_type=pl.DeviceIdType.MESH,
            )

        m2 = [recv_mirror(yq, k, s2_recv.at[k], s2_send.at[k]) for k in range(NCK)]
        m3 = [recv_mirror(zq, k, s3_recv.at[k], s3_send.at[k]) for k in range(NCK)]

        waited_m2, waited_m3 = set(), set()

        def gate(mirrors, waited, ks):
            for k in ks:
                if k not in waited:
                    mirrors[k].wait_recv()
                    waited.add(k)

        f4 = []
        for i, ((r0, nr), gates) in enumerate(FWD_Y):
            gate(m3, waited_m3, gates)
            r = pltpu.make_async_remote_copy(
                src_ref=out_ref.at[rowsr(zq, r0, nr), :],
                dst_ref=out_ref.at[rowsr(zq, r0, nr), :],
                send_sem=s4_send.at[i],
                recv_sem=s4_recv.at[i],
                device_id=yn,
                device_id_type=pl.DeviceIdType.MESH,
            )
            r.start()
            f4.append(r)

        f5 = []
        for i, ((r0, nr), gates) in enumerate(FWD_Z):
            gate(m2, waited_m2, gates)
            r = pltpu.make_async_remote_copy(
                src_ref=out_ref.at[rowsr(yq, r0, nr), :],
                dst_ref=out_ref.at[rowsr(yq, r0, nr), :],
                send_sem=s5_send.at[i],
                recv_sem=s5_recv.at[i],
                device_id=zn,
                device_id_type=pl.DeviceIdType.MESH,
            )
            r.start()
            f5.append(r)

        for k in range(NCK):
            if k not in waited_m2:
                m2[k].wait_recv()
            if k not in waited_m3:
                m3[k].wait_recv()

        def recv_mirror_rows(quarter, r0, nr, recv_sem, send_sem):
            return pltpu.make_async_remote_copy(
                src_ref=out_ref.at[rowsr(quarter, r0, nr), :],
                dst_ref=out_ref.at[rowsr(quarter, r0, nr), :],
                send_sem=send_sem,
                recv_sem=recv_sem,
                device_id=xn,
                device_id_type=pl.DeviceIdType.MESH,
            )

        for i, ((r0, nr), _) in enumerate(FWD_Y):
            recv_mirror_rows(dq, r0, nr, s4_recv.at[i], s4_send.at[i]).wait_recv()
        for i, ((r0, nr), _) in enumerate(FWD_Z):
            recv_mirror_rows(dq, r0, nr, s5_recv.at[i], s5_send.at[i]).wait_recv()
        for r in f1 + f2 + f3 + f4 + f5:
            r.wait_send()
        for cp in cps:
            cp.wait()

    return pl.pallas_call(
        body,
        out_shape=jax.ShapeDtypeStruct((m, n), x.dtype),
        in_specs=[pl.BlockSpec(memory_space=pltpu.VMEM)],
        out_specs=pl.BlockSpec(memory_space=pl.ANY),
        scratch_shapes=[
            pltpu.VMEM((QROWS, n), x.dtype),
            pltpu.VMEM((sum(nr for _, nr in DIAG_X), n), x.dtype),
            pltpu.SemaphoreType.DMA((NCK + NDIAG_X,)),
            pltpu.SemaphoreType.DMA((NCK + NDIAG_X,)),
            pltpu.SemaphoreType.DMA((NCK,)),
            pltpu.SemaphoreType.DMA((NCK,)),
            pltpu.SemaphoreType.DMA((NCK,)),
            pltpu.SemaphoreType.DMA((NCK,)),
            pltpu.SemaphoreType.DMA((len(FWD_Y),)),
            pltpu.SemaphoreType.DMA((len(FWD_Y),)),
            pltpu.SemaphoreType.DMA((len(FWD_Z),)),
            pltpu.SemaphoreType.DMA((len(FWD_Z),)),
            pltpu.SemaphoreType.DMA((NCK + NDIAG_X,)),
        ],
        compiler_params=pltpu.CompilerParams(
            collective_id=0,
            vmem_limit_bytes=56 * 1024 * 1024,
        ),
    )(x)
